# Initial kernel scaffold; baseline (speedup 1.0000x reference)
import jax
import jax.numpy as jnp
from jax.experimental import pallas as pl


def kernel(data, edge_attr, en_w1, en_b1, en_w2, en_b2, root_w, root_b, ni_w1, ni_b1, ni_w2, ni_b2, nj_w, nj_b, edge_index):
    raise NotImplementedError("write your pallas kernel here")


# speedup vs baseline: 2.4131x; 2.4131x over previous
import functools

import jax
import jax.numpy as jnp
from jax import lax
from jax.experimental import pallas as pl
from jax.experimental.pallas import tpu as pltpu
from jax.experimental.pallas import tpu_sc as plsc

N_NODES = 100000
N_EDGES = 3200000
FEAT = 19
HID = 9
GATHER = 64

NC = 2
NS = 16
COLS = 16
PER_T = N_EDGES // NS
CHUNK = 128
N_FULL = PER_T // CHUNK
TAIL = PER_T - N_FULL * CHUNK
N_PAD = 100352
STRIPE = N_PAD // NS
LANES = 16


def _softsign(x):
    return x / (1.0 + jnp.abs(x))


def _sc_body(data_hbm, src_hbm, dst_hbm, th_hbm, zin_hbm, out_hbm,
             agg, sidx, didx, thv, rows, sidx_t, didx_t, thv_t, rows_t,
             gsem):
    c = lax.axis_index("c")
    s = lax.axis_index("s")
    ebase = s * PER_T
    iota = lax.broadcasted_iota(jnp.int32, (LANES,), 0)
    k_vec = jnp.minimum(c * COLS + iota, FEAT - 1)

    zbase = s * STRIPE
    pltpu.sync_copy(zin_hbm, agg.at[pl.ds(zbase, STRIPE)])

    plsc.subcore_barrier()

    def do_chunk(e0, n, sidx_b, didx_b, thv_b, rows_b):
        pltpu.sync_copy(src_hbm.at[pl.ds(e0, n)], sidx_b)
        pltpu.async_copy(data_hbm.at[c].at[sidx_b], rows_b, gsem).wait()
        pltpu.sync_copy(th_hbm.at[pl.ds(e0, n)], thv_b)
        pltpu.sync_copy(dst_hbm.at[pl.ds(e0, n)], didx_b.at[0])

        @pl.loop(0, n)
        def _mul(i):
            th = plsc.load_gather(thv_b, [jnp.full((LANES,), i, jnp.int32),
                                          k_vec])
            rows_b[i] = rows_b[i] * th

        pltpu.sync_copy(rows_b, agg.at[didx_b.at[0]], add=True)

    @pl.loop(0, N_FULL)
    def _edges(i):
        e0 = pl.multiple_of(ebase + i * CHUNK, CHUNK)
        do_chunk(e0, CHUNK, sidx, didx, thv, rows)

    do_chunk(ebase + N_FULL * CHUNK, TAIL, sidx_t, didx_t, thv_t, rows_t)

    plsc.subcore_barrier()
    pltpu.sync_copy(agg.at[pl.ds(zbase, STRIPE)],
                    out_hbm.at[c, pl.ds(zbase, STRIPE)])


_sc_edge_pass = functools.partial(
    pl.kernel,
    out_type=jax.ShapeDtypeStruct((NC, N_PAD, COLS), jnp.float32),
    mesh=plsc.VectorSubcoreMesh(core_axis_name="c", subcore_axis_name="s",
                                num_cores=NC, num_subcores=NS),
    compiler_params=pltpu.CompilerParams(needs_layout_passes=False,
                                         use_tc_tiling_on_sc=False),
    scratch_types=[
        pltpu.VMEM_SHARED((N_PAD, COLS), jnp.float32),
        pltpu.VMEM((CHUNK,), jnp.int32),
        pltpu.VMEM((1, CHUNK), jnp.int32),
        pltpu.VMEM((CHUNK, FEAT), jnp.float32),
        pltpu.VMEM((CHUNK, COLS), jnp.float32),
        pltpu.VMEM((TAIL,), jnp.int32),
        pltpu.VMEM((1, TAIL), jnp.int32),
        pltpu.VMEM((TAIL, FEAT), jnp.float32),
        pltpu.VMEM((TAIL, COLS), jnp.float32),
        pltpu.SemaphoreType.DMA,
    ],
)(_sc_body)


def _tc_theta_body(ea_ref, w1_ref, b1_ref, w2_ref, b2_ref, o_ref):
    ea = ea_ref[...]
    h = _softsign(ea * w1_ref[...] + b1_ref[...])
    o_ref[...] = _softsign(
        jnp.dot(h, w2_ref[...], preferred_element_type=jnp.float32)
        + b2_ref[...])


def _tc_theta_pass(edge_attr, en_w1, en_b1, en_w2, en_b2):
    BE = 3200
    full = lambda shape: pl.BlockSpec(shape, lambda i: tuple(0 for _ in shape))
    return pl.pallas_call(
        _tc_theta_body,
        grid=(N_EDGES // BE,),
        in_specs=[
            pl.BlockSpec((BE, 1), lambda i: (i, 0)),
            full((1, HID)),
            full((1, HID)),
            full((HID, FEAT)),
            full((1, FEAT)),
        ],
        out_specs=pl.BlockSpec((BE, FEAT), lambda i: (i, 0)),
        out_shape=jax.ShapeDtypeStruct((N_EDGES, FEAT), jnp.float32),
    )(edge_attr, en_w1, en_b1.reshape(1, -1), en_w2, en_b2.reshape(1, -1))


def _tc_body(p_ref, x_ref, rw_ref, rb_ref, niw1a_ref, niw1b_ref, nib1_ref,
             niw2_ref, nib2_ref, njw_ref, njb_ref, o_ref):
    x = x_ref[...]
    agg = jnp.concatenate([p_ref[0], p_ref[1, :, :FEAT - COLS]], axis=1)
    h1 = agg + jnp.dot(x, rw_ref[...], preferred_element_type=jnp.float32)
    h1 = h1 + rb_ref[...]
    ni = jnp.dot(h1, niw1a_ref[...], preferred_element_type=jnp.float32)
    ni = ni + jnp.dot(x, niw1b_ref[...], preferred_element_type=jnp.float32)
    ni = _softsign(ni + nib1_ref[...])
    ni = _softsign(
        jnp.dot(ni, niw2_ref[...], preferred_element_type=jnp.float32)
        + nib2_ref[...])
    nj = _softsign(
        jnp.dot(x, njw_ref[...], preferred_element_type=jnp.float32)
        + njb_ref[...])
    o_ref[...] = jax.nn.softmax(ni, axis=1) * nj


def _tc_node_pass(partials, data, root_w, root_b, ni_w1a, ni_w1b, ni_b1,
                  ni_w2, ni_b2, nj_w, nj_b):
    R = 1000
    full = lambda shape: pl.BlockSpec(shape, lambda i: tuple(0 for _ in shape))
    return pl.pallas_call(
        _tc_body,
        grid=(N_NODES // R,),
        in_specs=[
            pl.BlockSpec((NC, R, COLS), lambda i: (0, i, 0)),
            pl.BlockSpec((R, FEAT), lambda i: (i, 0)),
            full((FEAT, FEAT)),
            full((1, FEAT)),
            full((FEAT, FEAT)),
            full((FEAT, FEAT)),
            full((1, FEAT)),
            full((FEAT, GATHER)),
            full((1, GATHER)),
            full((FEAT, GATHER)),
            full((1, GATHER)),
        ],
        out_specs=pl.BlockSpec((R, GATHER), lambda i: (i, 0)),
        out_shape=jax.ShapeDtypeStruct((N_NODES, GATHER), jnp.float32),
    )(partials, data, root_w, root_b, ni_w1a, ni_w1b, ni_b1, ni_w2, ni_b2,
      nj_w, nj_b)


def kernel(data, edge_attr, en_w1, en_b1, en_w2, en_b2, root_w, root_b,
           ni_w1, ni_b1, ni_w2, ni_b2, nj_w, nj_b, edge_index):
    src = edge_index[0].astype(jnp.int32)
    dst = edge_index[1].astype(jnp.int32)
    theta = _tc_theta_pass(edge_attr, en_w1, en_b1, en_w2, en_b2)
    data2 = jnp.stack([data[:, :COLS],
                       jnp.pad(data[:, COLS:],
                               ((0, 0), (0, 2 * COLS - FEAT)))])
    zin = jnp.zeros((STRIPE, COLS), jnp.float32)
    partials = _sc_edge_pass(data2, src, dst, theta, zin)
    return _tc_node_pass(
        partials, data, root_w, root_b.reshape(1, -1),
        ni_w1[:FEAT], ni_w1[FEAT:], ni_b1.reshape(1, -1),
        ni_w2, ni_b2.reshape(1, -1), nj_w, nj_b.reshape(1, -1))


# speedup vs baseline: 3.8482x; 1.5947x over previous
import functools

import jax
import jax.numpy as jnp
from jax import lax
from jax.experimental import pallas as pl
from jax.experimental.pallas import tpu as pltpu
from jax.experimental.pallas import tpu_sc as plsc

N_NODES = 100000
N_EDGES = 3200000
FEAT = 19
HID = 9
GATHER = 64

NC = 2
NS = 16
COLS = 16
PER_T = N_EDGES // NS
CHUNK = 128
N_FULL = PER_T // CHUNK
TAIL = PER_T - N_FULL * CHUNK
N_PAD = 100352
STRIPE = N_PAD // NS
LANES = 16


def _softsign(x):
    return x / (1.0 + jnp.abs(x))


def _sc_body(data_hbm, src_hbm, dst_hbm, th_hbm, zin_hbm, out_hbm,
             agg, sidx, didx, thv, rows, sidx_t, didx_t, thv_t, rows_t,
             ls0, ls1, ls2, gs0, gs1, gs2, ss0, ss1, ss2, gsem):
    c = lax.axis_index("c")
    s = lax.axis_index("s")
    ebase = s * PER_T
    iota = lax.broadcasted_iota(jnp.int32, (LANES,), 0)
    k_vec = jnp.minimum(c * COLS + iota, FEAT - 1)
    ls = [ls0, ls1, ls2]
    gs = [gs0, gs1, gs2]
    ss = [ss0, ss1, ss2]

    zbase = s * STRIPE
    pltpu.sync_copy(zin_hbm, agg.at[pl.ds(zbase, STRIPE)])

    plsc.subcore_barrier()

    def issue_loads(j, e0):
        pltpu.async_copy(src_hbm.at[pl.ds(e0, CHUNK)], sidx.at[j], ls[j])
        pltpu.async_copy(th_hbm.at[pl.ds(e0, CHUNK)], thv.at[j], ls[j])
        pltpu.async_copy(dst_hbm.at[pl.ds(e0, CHUNK)], didx.at[j], ls[j])

    def wait_loads(j):
        pltpu.make_async_copy(src_hbm.at[pl.ds(0, CHUNK)], sidx.at[j],
                              ls[j]).wait()
        pltpu.make_async_copy(th_hbm.at[pl.ds(0, CHUNK)], thv.at[j],
                              ls[j]).wait()
        pltpu.make_async_copy(dst_hbm.at[pl.ds(0, CHUNK)], didx.at[j],
                              ls[j]).wait()

    def issue_gather(j):
        pltpu.async_copy(data_hbm.at[c].at[sidx.at[j]], rows.at[j], gs[j])

    def wait_gather(j):
        pltpu.make_async_copy(data_hbm.at[c].at[sidx.at[j]], rows.at[j],
                              gs[j]).wait()

    def issue_scatter(j):
        pltpu.async_copy(rows.at[j], agg.at[didx.at[j]], ss[j], add=True)

    def wait_scatter(j):
        pltpu.make_async_copy(rows.at[j], agg.at[didx.at[j]], ss[j]).wait()

    def multiply(j):
        @pl.loop(0, CHUNK, step=8)
        def _m(i0):
            for u in range(8):
                i_s = jnp.full((LANES,), i0 + u, jnp.int32)
                th = plsc.load_gather(thv.at[j], [i_s, k_vec])
                r = plsc.load_gather(rows.at[j], [i_s, iota])
                plsc.store_scatter(rows.at[j], [i_s, iota], r * th)

    MAIN = (N_FULL // 3) * 3

    issue_loads(0, ebase)
    issue_loads(1, ebase + CHUNK)
    wait_loads(0)
    issue_gather(0)

    @pl.loop(0, MAIN, step=3)
    def _edges(base):
        for j in range(3):
            i = base + j
            b = (j + 1) % 3
            cc = (j + 2) % 3
            wait_loads(b)

            @pl.when(i > 0)
            def _w():
                wait_scatter(cc)

            issue_gather(b)
            issue_loads(cc, ebase + (i + 2) * CHUNK)
            wait_gather(j)
            multiply(j)
            issue_scatter(j)

    wait_loads(1)
    wait_scatter(2)
    issue_gather(1)
    wait_gather(0)
    multiply(0)
    issue_scatter(0)
    wait_gather(1)
    multiply(1)
    issue_scatter(1)

    e0 = ebase + N_FULL * CHUNK
    pltpu.sync_copy(src_hbm.at[pl.ds(e0, TAIL)], sidx_t)
    pltpu.async_copy(data_hbm.at[c].at[sidx_t], rows_t, gsem).wait()
    pltpu.sync_copy(th_hbm.at[pl.ds(e0, TAIL)], thv_t)
    pltpu.sync_copy(dst_hbm.at[pl.ds(e0, TAIL)], didx_t.at[0])

    @pl.loop(0, TAIL, step=8)
    def _mt(i0):
        for u in range(8):
            i_s = jnp.full((LANES,), i0 + u, jnp.int32)
            th = plsc.load_gather(thv_t, [i_s, k_vec])
            r = plsc.load_gather(rows_t, [i_s, iota])
            plsc.store_scatter(rows_t, [i_s, iota], r * th)

    pltpu.sync_copy(rows_t, agg.at[didx_t.at[0]], add=True)

    wait_scatter(0)
    wait_scatter(1)

    plsc.subcore_barrier()
    pltpu.sync_copy(agg.at[pl.ds(zbase, STRIPE)],
                    out_hbm.at[c, pl.ds(zbase, STRIPE)])


_sc_edge_pass = functools.partial(
    pl.kernel,
    out_type=jax.ShapeDtypeStruct((NC, N_PAD, COLS), jnp.float32),
    mesh=plsc.VectorSubcoreMesh(core_axis_name="c", subcore_axis_name="s",
                                num_cores=NC, num_subcores=NS),
    compiler_params=pltpu.CompilerParams(needs_layout_passes=False,
                                         use_tc_tiling_on_sc=False),
    scratch_types=[
        pltpu.VMEM_SHARED((N_PAD, COLS), jnp.float32),
        pltpu.VMEM((3, CHUNK), jnp.int32),
        pltpu.VMEM((3, CHUNK), jnp.int32),
        pltpu.VMEM((3, CHUNK, FEAT), jnp.float32),
        pltpu.VMEM((3, CHUNK, COLS), jnp.float32),
        pltpu.VMEM((TAIL,), jnp.int32),
        pltpu.VMEM((1, TAIL), jnp.int32),
        pltpu.VMEM((TAIL, FEAT), jnp.float32),
        pltpu.VMEM((TAIL, COLS), jnp.float32),
    ] + [pltpu.SemaphoreType.DMA] * 10,
)(_sc_body)


def _tc_theta_body(ea_ref, w1_ref, b1_ref, w2_ref, b2_ref, o_ref):
    ea = ea_ref[...]
    h = _softsign(ea * w1_ref[...] + b1_ref[...])
    o_ref[...] = _softsign(
        jnp.dot(h, w2_ref[...], preferred_element_type=jnp.float32)
        + b2_ref[...])


def _tc_theta_pass(edge_attr, en_w1, en_b1, en_w2, en_b2):
    BE = 3200
    full = lambda shape: pl.BlockSpec(shape, lambda i: tuple(0 for _ in shape))
    return pl.pallas_call(
        _tc_theta_body,
        grid=(N_EDGES // BE,),
        in_specs=[
            pl.BlockSpec((BE, 1), lambda i: (i, 0)),
            full((1, HID)),
            full((1, HID)),
            full((HID, FEAT)),
            full((1, FEAT)),
        ],
        out_specs=pl.BlockSpec((BE, FEAT), lambda i: (i, 0)),
        out_shape=jax.ShapeDtypeStruct((N_EDGES, FEAT), jnp.float32),
    )(edge_attr, en_w1, en_b1.reshape(1, -1), en_w2, en_b2.reshape(1, -1))


def _tc_body(p_ref, x_ref, rw_ref, rb_ref, niw1a_ref, niw1b_ref, nib1_ref,
             niw2_ref, nib2_ref, njw_ref, njb_ref, o_ref):
    x = x_ref[...]
    agg = jnp.concatenate([p_ref[0], p_ref[1, :, :FEAT - COLS]], axis=1)
    h1 = agg + jnp.dot(x, rw_ref[...], preferred_element_type=jnp.float32)
    h1 = h1 + rb_ref[...]
    ni = jnp.dot(h1, niw1a_ref[...], preferred_element_type=jnp.float32)
    ni = ni + jnp.dot(x, niw1b_ref[...], preferred_element_type=jnp.float32)
    ni = _softsign(ni + nib1_ref[...])
    ni = _softsign(
        jnp.dot(ni, niw2_ref[...], preferred_element_type=jnp.float32)
        + nib2_ref[...])
    nj = _softsign(
        jnp.dot(x, njw_ref[...], preferred_element_type=jnp.float32)
        + njb_ref[...])
    o_ref[...] = jax.nn.softmax(ni, axis=1) * nj


def _tc_node_pass(partials, data, root_w, root_b, ni_w1a, ni_w1b, ni_b1,
                  ni_w2, ni_b2, nj_w, nj_b):
    R = 1000
    full = lambda shape: pl.BlockSpec(shape, lambda i: tuple(0 for _ in shape))
    return pl.pallas_call(
        _tc_body,
        grid=(N_NODES // R,),
        in_specs=[
            pl.BlockSpec((NC, R, COLS), lambda i: (0, i, 0)),
            pl.BlockSpec((R, FEAT), lambda i: (i, 0)),
            full((FEAT, FEAT)),
            full((1, FEAT)),
            full((FEAT, FEAT)),
            full((FEAT, FEAT)),
            full((1, FEAT)),
            full((FEAT, GATHER)),
            full((1, GATHER)),
            full((FEAT, GATHER)),
            full((1, GATHER)),
        ],
        out_specs=pl.BlockSpec((R, GATHER), lambda i: (i, 0)),
        out_shape=jax.ShapeDtypeStruct((N_NODES, GATHER), jnp.float32),
    )(partials, data, root_w, root_b, ni_w1a, ni_w1b, ni_b1, ni_w2, ni_b2,
      nj_w, nj_b)


def kernel(data, edge_attr, en_w1, en_b1, en_w2, en_b2, root_w, root_b,
           ni_w1, ni_b1, ni_w2, ni_b2, nj_w, nj_b, edge_index):
    src = edge_index[0].astype(jnp.int32)
    dst = edge_index[1].astype(jnp.int32)
    theta = _tc_theta_pass(edge_attr, en_w1, en_b1, en_w2, en_b2)
    data2 = jnp.stack([data[:, :COLS],
                       jnp.pad(data[:, COLS:],
                               ((0, 0), (0, 2 * COLS - FEAT)))])
    zin = jnp.zeros((STRIPE, COLS), jnp.float32)
    partials = _sc_edge_pass(data2, src, dst, theta, zin)
    return _tc_node_pass(
        partials, data, root_w, root_b.reshape(1, -1),
        ni_w1[:FEAT], ni_w1[FEAT:], ni_b1.reshape(1, -1),
        ni_w2, ni_b2.reshape(1, -1), nj_w, nj_b.reshape(1, -1))
